# baseline (speedup 1.0000x reference)
import functools

import jax
import jax.numpy as jnp
from jax import lax
from jax.experimental import pallas as pl
from jax.experimental.pallas import tpu as pltpu
from jax.experimental.pallas import tpu_sc as plsc

V = 1_000_000
D = 32
N_LEVELS = 127.0
_MAGIC = 1.5 * 2.0 ** 23

_CUT = 999_936
_TAIL = V - _CUT
_TROWS = V * D // 128
_TAILR = _TAIL * D // 128

_SC_ = 512
_NSLAB = _CUT // _SC_
_GRID_S = _NSLAB + 1


def _slab_body(wt_ref, slab_ref, scale_ref, svec_ref, ivec_ref, acc_ref):
    i = pl.program_id(0)
    w = wt_ref[...]
    slab_ref[0] = w
    col = i * _SC_ + lax.broadcasted_iota(jnp.int32, (D, _SC_), 1)
    m = jnp.max(jnp.where(col < V, jnp.abs(w), 0.0))

    @pl.when(i == 0)
    def _():
        acc_ref[0] = m

    @pl.when(i > 0)
    def _():
        acc_ref[0] = jnp.maximum(acc_ref[0], m)

    @pl.when(i == _GRID_S - 1)
    def _():
        s = jnp.maximum(acc_ref[0], 1e-8) / N_LEVELS
        scale_ref[0] = s
        for k in range(16):
            svec_ref[k] = s
            ivec_ref[k] = 1.0 / s


_slab_call = pl.pallas_call(
    _slab_body,
    grid=(_GRID_S,),
    in_specs=[pl.BlockSpec((D, _SC_), lambda i: (0, i))],
    out_specs=[
        pl.BlockSpec((1, D, _SC_), lambda i: (i, 0, 0)),
        pl.BlockSpec(memory_space=pltpu.SMEM),
        pl.BlockSpec(memory_space=pltpu.SMEM),
        pl.BlockSpec(memory_space=pltpu.SMEM),
    ],
    out_shape=[
        jax.ShapeDtypeStruct((_GRID_S, D, _SC_), jnp.float32),
        jax.ShapeDtypeStruct((1,), jnp.float32),
        jax.ShapeDtypeStruct((16,), jnp.float32),
        jax.ShapeDtypeStruct((16,), jnp.float32),
    ],
    scratch_shapes=[pltpu.SMEM((1,), jnp.float32)],
)

_C61 = 16_384
_TOFF = _CUT - 61 * _C61


def _tail_body(scale_ref, wt_ref, out_ref):
    s = scale_ref[0]
    t = wt_ref[:, _TOFF:_TOFF + _TAIL]
    t = jnp.clip(jnp.round(t / s), -N_LEVELS, N_LEVELS - 1.0) * s
    r = lax.broadcasted_iota(jnp.int32, (_TAILR, _TAIL), 0)
    c = lax.broadcasted_iota(jnp.int32, (_TAILR, _TAIL), 1)
    outs = []
    for k in range(4):
        g = (c == 4 * r + k).astype(jnp.float32)
        outs.append(
            lax.dot_general(g, t, (((1,), (1,)), ((), ())),
                            preferred_element_type=jnp.float32))
    out_ref[...] = jnp.concatenate(outs, axis=1)


_tail_call = pl.pallas_call(
    _tail_body,
    grid=(1,),
    in_specs=[
        pl.BlockSpec(memory_space=pltpu.SMEM),
        pl.BlockSpec((D, _C61), lambda i: (0, 61)),
    ],
    out_specs=pl.BlockSpec((_TAILR, 128), lambda i: (0, 0)),
    out_shape=jax.ShapeDtypeStruct((_TAILR, 128), jnp.float32),
)

_NC, _NS = 2, 16
_NW = _NC * _NS
_OR = _SC_ * D // 128
_CPW = -(-_NSLAB // _NW)


def _transpose_body(slab_ref, tail_ref, svec_ref, ivec_ref, out_ref,
                    in_v, out_v, tail_v, sv_v, iv_v, semi, semo):
    cc = lax.axis_index("c")
    ss = lax.axis_index("s")
    wid = ss * _NC + cc
    iota = lax.iota(jnp.int32, 16)

    pltpu.sync_copy(svec_ref, sv_v)
    pltpu.sync_copy(ivec_ref, iv_v)
    svec = sv_v[...]
    ivec = iv_v[...]

    def slab_of(slot):
        return slot * _NW + wid

    def in_copy(slot, p):
        return pltpu.make_async_copy(
            slab_ref.at[slab_of(slot)], in_v.at[p], semi)

    def out_copy(slot, p):
        return pltpu.make_async_copy(
            out_v.at[p], out_ref.at[slab_of(slot)], semo)

    in_copy(0, 0).start()

    @pl.when(wid == 0)
    def _():
        pltpu.sync_copy(tail_ref, tail_v)
        pltpu.sync_copy(tail_v, out_ref.at[_NSLAB, pl.ds(0, _TAILR), :])

    def slot_body(slot, carry):
        p = lax.rem(slot, 2)

        @pl.when(slab_of(slot + 1) < _NSLAB)
        def _():
            in_copy(slot + 1, 1 - p).start()

        @pl.when(jnp.logical_and(slot >= 2, slab_of(slot - 2) < _NSLAB))
        def _():
            out_copy(slot - 2, p).wait()

        @pl.when(slab_of(slot) < _NSLAB)
        def _():
            in_copy(slot, p).wait()

            def body(g, c2):
                for d in range(D):
                    v = in_v[p, d, pl.ds(g * 16, 16)]
                    q = (v * ivec + _MAGIC) - _MAGIC
                    vq = jnp.minimum(q, N_LEVELS - 1.0) * svec
                    flat = 512 * g + 32 * iota + d
                    plsc.store_scatter(
                        out_v.at[p],
                        [lax.shift_right_logical(flat, 7),
                         lax.bitwise_and(flat, 127)],
                        vq,
                    )
                return c2

            lax.fori_loop(0, _SC_ // 16, body, 0, unroll=2)
            out_copy(slot, p).start()

        return carry

    lax.fori_loop(0, _CPW, slot_body, 0)

    for s in (_CPW - 2, _CPW - 1):

        @pl.when(slab_of(s) < _NSLAB)
        def _(s=s):
            out_copy(s, s % 2).wait()


_transpose_call = functools.partial(
    pl.kernel,
    mesh=plsc.VectorSubcoreMesh(
        core_axis_name="c", subcore_axis_name="s", num_cores=_NC, num_subcores=_NS
    ),
    out_type=jax.ShapeDtypeStruct((_GRID_S, _OR, 128), jnp.float32),
    scratch_types=[
        pltpu.VMEM((2, D, _SC_), jnp.float32),
        pltpu.VMEM((2, _OR, 128), jnp.float32),
        pltpu.VMEM((_TAILR, 128), jnp.float32),
        pltpu.VMEM((16,), jnp.float32),
        pltpu.VMEM((16,), jnp.float32),
        pltpu.SemaphoreType.DMA,
        pltpu.SemaphoreType.DMA,
    ],
    compiler_params=pltpu.CompilerParams(
        use_tc_tiling_on_sc=True, needs_layout_passes=False
    ),
)(_transpose_body)

_B = 16384 * 20
_BPW = _B // _NW
_CH = 128
_K = 8
_GCH = _CH * _K
_NG = _BPW // _GCH


def _gather_body(x_ref, w_ref, out_ref, idx_v, rows_v, semg, semw):
    cc = lax.axis_index("c")
    ss = lax.axis_index("s")
    wid = ss * _NC + cc
    base = wid * _BPW
    pltpu.sync_copy(x_ref.at[pl.ds(base, _BPW)], idx_v)
    writes = [None, None]
    for g in range(_NG):
        p = g % 2
        if writes[p] is not None:
            writes[p].wait()
        descs = [
            pltpu.async_copy(
                w_ref.at[idx_v.at[pl.ds(g * _GCH + j * _CH, _CH)]],
                rows_v.at[p, pl.ds(j * _CH, _CH)],
                semg,
            )
            for j in range(_K)
        ]
        for d_ in descs:
            d_.wait()
        writes[p] = pltpu.async_copy(
            rows_v.at[p], out_ref.at[pl.ds(base + g * _GCH, _GCH)], semw
        )
    for wdesc in writes:
        if wdesc is not None:
            wdesc.wait()


_gather_call = functools.partial(
    pl.kernel,
    mesh=plsc.VectorSubcoreMesh(
        core_axis_name="c", subcore_axis_name="s", num_cores=_NC, num_subcores=_NS
    ),
    out_type=jax.ShapeDtypeStruct((_B, D), jnp.float32),
    scratch_types=[
        pltpu.VMEM((_BPW,), jnp.int32),
        pltpu.VMEM((2, _GCH, D), jnp.float32),
        pltpu.SemaphoreType.DMA,
        pltpu.SemaphoreType.DMA,
    ],
    compiler_params=pltpu.CompilerParams(use_tc_tiling_on_sc=False),
)(_gather_body)


def kernel(weight, x):
    wt = weight.T
    slabs, scale, svec, ivec = _slab_call(wt)
    tail = _tail_call(scale, wt)
    table = _transpose_call(slabs, tail, svec, ivec)
    tview = table.reshape(_GRID_S * _OR * 128 // D, D)
    xf = x.reshape(-1)
    out = _gather_call(xf, tview)
    return out.reshape(x.shape + (D,)), scale


# speedup vs baseline: 1.9025x; 1.9025x over previous
import functools

import jax
import jax.numpy as jnp
from jax import lax
from jax.experimental import pallas as pl
from jax.experimental.pallas import tpu as pltpu
from jax.experimental.pallas import tpu_sc as plsc

V = 1_000_000
D = 32
N_LEVELS = 127.0
_MAGIC = 1.5 * 2.0 ** 23

_CUT = 999_936
_TAIL = V - _CUT
_TROWS = V * D // 128
_TAILR = _TAIL * D // 128

_SC_ = 512
_NSLAB = _CUT // _SC_
_SPB = 4
_GRID_B = 489
_NSLABT = _GRID_B * _SPB
_GRID_S = _NSLAB + 1


def _slab_body(wt_ref, slab_ref, scale_ref, svec_ref, ivec_ref, acc_ref):
    i = pl.program_id(0)
    w = wt_ref[...]
    for k in range(_SPB):
        slab_ref[k] = w[:, k * _SC_:(k + 1) * _SC_]
    col = i * (_SPB * _SC_) + lax.broadcasted_iota(
        jnp.int32, (D, _SPB * _SC_), 1)
    m = jnp.max(jnp.where(col < V, jnp.abs(w), 0.0))

    @pl.when(i == 0)
    def _():
        acc_ref[0] = m

    @pl.when(i > 0)
    def _():
        acc_ref[0] = jnp.maximum(acc_ref[0], m)

    @pl.when(i == _GRID_B - 1)
    def _():
        s = jnp.maximum(acc_ref[0], 1e-8) / N_LEVELS
        scale_ref[0] = s
        for k in range(16):
            svec_ref[k] = s
            ivec_ref[k] = 1.0 / s


_slab_call = pl.pallas_call(
    _slab_body,
    grid=(_GRID_B,),
    in_specs=[pl.BlockSpec((D, _SPB * _SC_), lambda i: (0, i))],
    out_specs=[
        pl.BlockSpec((_SPB, D, _SC_), lambda i: (i, 0, 0)),
        pl.BlockSpec(memory_space=pltpu.SMEM),
        pl.BlockSpec(memory_space=pltpu.SMEM),
        pl.BlockSpec(memory_space=pltpu.SMEM),
    ],
    out_shape=[
        jax.ShapeDtypeStruct((_NSLABT, D, _SC_), jnp.float32),
        jax.ShapeDtypeStruct((1,), jnp.float32),
        jax.ShapeDtypeStruct((16,), jnp.float32),
        jax.ShapeDtypeStruct((16,), jnp.float32),
    ],
    scratch_shapes=[pltpu.SMEM((1,), jnp.float32)],
)

_C61 = 16_384
_TOFF = _CUT - 61 * _C61


def _tail_body(wt_ref, out_ref):
    t = wt_ref[:, _TOFF:_TOFF + _TAIL]
    r = lax.broadcasted_iota(jnp.int32, (_TAILR, _TAIL), 0)
    c = lax.broadcasted_iota(jnp.int32, (_TAILR, _TAIL), 1)
    outs = []
    for k in range(4):
        g = (c == 4 * r + k).astype(jnp.float32)
        outs.append(
            lax.dot_general(g, t, (((1,), (1,)), ((), ())),
                            preferred_element_type=jnp.float32))
    out_ref[...] = jnp.concatenate(outs, axis=1)


_tail_call = pl.pallas_call(
    _tail_body,
    grid=(1,),
    in_specs=[pl.BlockSpec((D, _C61), lambda i: (0, 61))],
    out_specs=pl.BlockSpec((_TAILR, 128), lambda i: (0, 0)),
    out_shape=jax.ShapeDtypeStruct((_TAILR, 128), jnp.float32),
)

_NC, _NS = 2, 16
_NW = _NC * _NS
_OR = _SC_ * D // 128
_CPW = -(-_NSLAB // _NW)


def _transpose_body(slab_ref, tail_ref, out_ref, in_v, out_v, tail_v,
                    semi, semo):
    cc = lax.axis_index("c")
    ss = lax.axis_index("s")
    wid = ss * _NC + cc
    iota = lax.iota(jnp.int32, 16)
    rowbase = lax.shift_right_logical(iota, 2)
    colbase = 32 * lax.bitwise_and(iota, 3)

    def slab_of(slot):
        return slot * _NW + wid

    def in_copy(slot, p):
        return pltpu.make_async_copy(
            slab_ref.at[slab_of(slot)], in_v.at[p], semi)

    def out_copy(slot, p):
        return pltpu.make_async_copy(
            out_v.at[p], out_ref.at[slab_of(slot)], semo)

    in_copy(0, 0).start()

    @pl.when(wid == 0)
    def _():
        pltpu.sync_copy(tail_ref, tail_v)
        pltpu.sync_copy(tail_v, out_ref.at[_NSLAB, pl.ds(0, _TAILR), :])

    def slot_body(slot, carry):
        p = lax.rem(slot, 2)

        @pl.when(slab_of(slot + 1) < _NSLAB)
        def _():
            in_copy(slot + 1, 1 - p).start()

        @pl.when(jnp.logical_and(slot >= 2, slab_of(slot - 2) < _NSLAB))
        def _():
            out_copy(slot - 2, p).wait()

        @pl.when(slab_of(slot) < _NSLAB)
        def _():
            in_copy(slot, p).wait()
            for d in range(D):
                col = colbase + d

                def body(g, c2):
                    v = in_v[p, d, pl.ds(g * 16, 16)]
                    plsc.store_scatter(
                        out_v.at[p], [rowbase + 4 * g, col], v)
                    return c2

                lax.fori_loop(0, _SC_ // 16, body, 0, unroll=8)
            out_copy(slot, p).start()

        return carry

    lax.fori_loop(0, _CPW, slot_body, 0)

    for s in (_CPW - 2, _CPW - 1):

        @pl.when(slab_of(s) < _NSLAB)
        def _(s=s):
            out_copy(s, s % 2).wait()


_transpose_call = functools.partial(
    pl.kernel,
    mesh=plsc.VectorSubcoreMesh(
        core_axis_name="c", subcore_axis_name="s", num_cores=_NC, num_subcores=_NS
    ),
    out_type=jax.ShapeDtypeStruct((_GRID_S, _OR, 128), jnp.float32),
    scratch_types=[
        pltpu.VMEM((2, D, _SC_), jnp.float32),
        pltpu.VMEM((2, _OR, 128), jnp.float32),
        pltpu.VMEM((_TAILR, 128), jnp.float32),
        pltpu.SemaphoreType.DMA,
        pltpu.SemaphoreType.DMA,
    ],
    compiler_params=pltpu.CompilerParams(
        use_tc_tiling_on_sc=True, needs_layout_passes=False
    ),
)(_transpose_body)

_B = 16384 * 20
_BPW = _B // _NW
_CH = 128
_K = 8
_GCH = _CH * _K
_NG = _BPW // _GCH


def _gather_body(x_ref, w_ref, svec_ref, ivec_ref, out_ref,
                 idx_v, rows_v, sv_v, iv_v, semg, semw):
    cc = lax.axis_index("c")
    ss = lax.axis_index("s")
    wid = ss * _NC + cc
    base = wid * _BPW
    pltpu.sync_copy(svec_ref, sv_v)
    pltpu.sync_copy(ivec_ref, iv_v)
    svec = sv_v[...]
    ivec = iv_v[...]
    pltpu.sync_copy(x_ref.at[pl.ds(base, _BPW)], idx_v)

    def fire(g, p):
        return [
            pltpu.async_copy(
                w_ref.at[idx_v.at[pl.ds(g * _GCH + j * _CH, _CH)]],
                rows_v.at[p, pl.ds(j * _CH, _CH)],
                semg,
            )
            for j in range(_K)
        ]

    descs = fire(0, 0)
    writes = [None, None]
    for g in range(_NG):
        p = g % 2
        for d_ in descs:
            d_.wait()
        if g + 1 < _NG:
            if writes[1 - p] is not None:
                writes[1 - p].wait()
            descs = fire(g + 1, 1 - p)

        def qbody(r, c2):
            for h in (0, 1):
                v = rows_v[p, r, pl.ds(16 * h, 16)]
                q = (v * ivec + _MAGIC) - _MAGIC
                rows_v[p, r, pl.ds(16 * h, 16)] = (
                    jnp.minimum(q, N_LEVELS - 1.0) * svec)
            return c2

        lax.fori_loop(0, _GCH, qbody, 0, unroll=4)
        writes[p] = pltpu.async_copy(
            rows_v.at[p], out_ref.at[pl.ds(base + g * _GCH, _GCH)], semw
        )
    for wdesc in writes:
        if wdesc is not None:
            wdesc.wait()


_gather_call = functools.partial(
    pl.kernel,
    mesh=plsc.VectorSubcoreMesh(
        core_axis_name="c", subcore_axis_name="s", num_cores=_NC, num_subcores=_NS
    ),
    out_type=jax.ShapeDtypeStruct((_B, D), jnp.float32),
    scratch_types=[
        pltpu.VMEM((_BPW,), jnp.int32),
        pltpu.VMEM((2, _GCH, D), jnp.float32),
        pltpu.VMEM((16,), jnp.float32),
        pltpu.VMEM((16,), jnp.float32),
        pltpu.SemaphoreType.DMA,
        pltpu.SemaphoreType.DMA,
    ],
    compiler_params=pltpu.CompilerParams(use_tc_tiling_on_sc=False),
)(_gather_body)


def kernel(weight, x):
    wt = weight.T
    slabs, scale, svec, ivec = _slab_call(wt)
    tail = _tail_call(wt)
    table = _transpose_call(slabs, tail)
    tview = table.reshape(_GRID_S * _OR * 128 // D, D)
    xf = x.reshape(-1)
    out = _gather_call(xf, tview, svec, ivec)
    return out.reshape(x.shape + (D,)), scale


# speedup vs baseline: 1.9241x; 1.0114x over previous
import functools

import jax
import jax.numpy as jnp
from jax import lax
from jax.experimental import pallas as pl
from jax.experimental.pallas import tpu as pltpu
from jax.experimental.pallas import tpu_sc as plsc

V = 1_000_000
D = 32
N_LEVELS = 127.0
_MAGIC = 1.5 * 2.0 ** 23

_CUT = 999_936
_TAIL = V - _CUT
_TROWS = V * D // 128
_TAILR = _TAIL * D // 128

_SC_ = 512
_NSLAB = _CUT // _SC_
_SPB = 4
_GRID_B = 489
_NSLABT = _GRID_B * _SPB
_GRID_S = _NSLAB + 1


def _slab_body(wt_ref, slab_ref, scale_ref, svec_ref, ivec_ref, acc_ref):
    i = pl.program_id(0)
    w = wt_ref[...]
    for k in range(_SPB):
        slab_ref[k] = w[:, k * _SC_:(k + 1) * _SC_]

    @pl.when(i == 0)
    def _():
        acc_ref[0] = jnp.max(jnp.abs(w))

    @pl.when(jnp.logical_and(i > 0, i < _GRID_B - 1))
    def _():
        acc_ref[0] = jnp.maximum(acc_ref[0], jnp.max(jnp.abs(w)))

    @pl.when(i == _GRID_B - 1)
    def _():
        col = i * (_SPB * _SC_) + lax.broadcasted_iota(
            jnp.int32, (D, _SPB * _SC_), 1)
        m = jnp.max(jnp.where(col < V, jnp.abs(w), 0.0))
        s = jnp.maximum(jnp.maximum(acc_ref[0], m), 1e-8) / N_LEVELS
        scale_ref[0] = s
        for k in range(16):
            svec_ref[k] = s
            ivec_ref[k] = 1.0 / s


_slab_call = pl.pallas_call(
    _slab_body,
    grid=(_GRID_B,),
    in_specs=[pl.BlockSpec((D, _SPB * _SC_), lambda i: (0, i))],
    out_specs=[
        pl.BlockSpec((_SPB, D, _SC_), lambda i: (i, 0, 0)),
        pl.BlockSpec(memory_space=pltpu.SMEM),
        pl.BlockSpec(memory_space=pltpu.SMEM),
        pl.BlockSpec(memory_space=pltpu.SMEM),
    ],
    out_shape=[
        jax.ShapeDtypeStruct((_NSLABT, D, _SC_), jnp.float32),
        jax.ShapeDtypeStruct((1,), jnp.float32),
        jax.ShapeDtypeStruct((16,), jnp.float32),
        jax.ShapeDtypeStruct((16,), jnp.float32),
    ],
    scratch_shapes=[pltpu.SMEM((1,), jnp.float32)],
)

_C61 = 16_384
_TOFF = _CUT - 61 * _C61


def _tail_body(wt_ref, out_ref):
    t = wt_ref[:, _TOFF:_TOFF + _TAIL]
    r = lax.broadcasted_iota(jnp.int32, (_TAILR, _TAIL), 0)
    c = lax.broadcasted_iota(jnp.int32, (_TAILR, _TAIL), 1)
    outs = []
    for k in range(4):
        g = (c == 4 * r + k).astype(jnp.float32)
        outs.append(
            lax.dot_general(g, t, (((1,), (1,)), ((), ())),
                            preferred_element_type=jnp.float32))
    out_ref[...] = jnp.concatenate(outs, axis=1)


_tail_call = pl.pallas_call(
    _tail_body,
    grid=(1,),
    in_specs=[pl.BlockSpec((D, _C61), lambda i: (0, 61))],
    out_specs=pl.BlockSpec((_TAILR, 128), lambda i: (0, 0)),
    out_shape=jax.ShapeDtypeStruct((_TAILR, 128), jnp.float32),
)

_NC, _NS = 2, 16
_NW = _NC * _NS
_OR = _SC_ * D // 128
_CPW = -(-_NSLAB // _NW)


def _transpose_body(slab_ref, tail_ref, out_ref, in_v, out_v, tail_v,
                    semi, semo):
    cc = lax.axis_index("c")
    ss = lax.axis_index("s")
    wid = ss * _NC + cc
    iota = lax.iota(jnp.int32, 16)
    rowbase = lax.shift_right_logical(iota, 2)
    colbase = 32 * lax.bitwise_and(iota, 3)

    def slab_of(slot):
        return slot * _NW + wid

    def in_copy(slot, p):
        return pltpu.make_async_copy(
            slab_ref.at[slab_of(slot)], in_v.at[p], semi)

    def out_copy(slot, p):
        return pltpu.make_async_copy(
            out_v.at[p], out_ref.at[slab_of(slot)], semo)

    in_copy(0, 0).start()

    @pl.when(wid == 0)
    def _():
        pltpu.sync_copy(tail_ref, tail_v)
        pltpu.sync_copy(tail_v, out_ref.at[_NSLAB, pl.ds(0, _TAILR), :])

    def slot_body(slot, carry):
        p = lax.rem(slot, 2)

        @pl.when(slab_of(slot + 1) < _NSLAB)
        def _():
            in_copy(slot + 1, 1 - p).start()

        @pl.when(jnp.logical_and(slot >= 2, slab_of(slot - 2) < _NSLAB))
        def _():
            out_copy(slot - 2, p).wait()

        @pl.when(slab_of(slot) < _NSLAB)
        def _():
            in_copy(slot, p).wait()
            for d in range(D):
                col = colbase + d

                @plsc.parallel_loop(0, _SC_ // 16, unroll=8)
                def _(g, d=d, col=col):
                    v = in_v[p, d, pl.ds(g * 16, 16)]
                    plsc.store_scatter(
                        out_v.at[p], [rowbase + 4 * g, col], v)
            out_copy(slot, p).start()

        return carry

    lax.fori_loop(0, _CPW, slot_body, 0)

    for s in (_CPW - 2, _CPW - 1):

        @pl.when(slab_of(s) < _NSLAB)
        def _(s=s):
            out_copy(s, s % 2).wait()


_transpose_call = functools.partial(
    pl.kernel,
    mesh=plsc.VectorSubcoreMesh(
        core_axis_name="c", subcore_axis_name="s", num_cores=_NC, num_subcores=_NS
    ),
    out_type=jax.ShapeDtypeStruct((_GRID_S, _OR, 128), jnp.float32),
    scratch_types=[
        pltpu.VMEM((2, D, _SC_), jnp.float32),
        pltpu.VMEM((2, _OR, 128), jnp.float32),
        pltpu.VMEM((_TAILR, 128), jnp.float32),
        pltpu.SemaphoreType.DMA,
        pltpu.SemaphoreType.DMA,
    ],
    compiler_params=pltpu.CompilerParams(
        use_tc_tiling_on_sc=True, needs_layout_passes=False
    ),
)(_transpose_body)

_B = 16384 * 20
_BPW = _B // _NW
_CH = 128
_K = 8
_GCH = _CH * _K
_NG = _BPW // _GCH


def _gather_body(x_ref, w_ref, svec_ref, ivec_ref, out_ref,
                 idx_v, rows_v, sv_v, iv_v, semg, semw):
    cc = lax.axis_index("c")
    ss = lax.axis_index("s")
    wid = ss * _NC + cc
    base = wid * _BPW
    pltpu.sync_copy(svec_ref, sv_v)
    pltpu.sync_copy(ivec_ref, iv_v)
    svec = sv_v[...]
    ivec = iv_v[...]
    pltpu.sync_copy(x_ref.at[pl.ds(base, _BPW)], idx_v)

    def fire(g, p):
        return [
            pltpu.async_copy(
                w_ref.at[idx_v.at[pl.ds(g * _GCH + j * _CH, _CH)]],
                rows_v.at[p, pl.ds(j * _CH, _CH)],
                semg,
            )
            for j in range(_K)
        ]

    descs = fire(0, 0)
    writes = [None, None]
    for g in range(_NG):
        p = g % 2
        for d_ in descs:
            d_.wait()
        if g + 1 < _NG:
            if writes[1 - p] is not None:
                writes[1 - p].wait()
            descs = fire(g + 1, 1 - p)

        def qbody(r, c2):
            for h in (0, 1):
                v = rows_v[p, r, pl.ds(16 * h, 16)]
                q = (v * ivec + _MAGIC) - _MAGIC
                rows_v[p, r, pl.ds(16 * h, 16)] = (
                    jnp.minimum(q, N_LEVELS - 1.0) * svec)
            return c2

        lax.fori_loop(0, _GCH, qbody, 0, unroll=4)
        writes[p] = pltpu.async_copy(
            rows_v.at[p], out_ref.at[pl.ds(base + g * _GCH, _GCH)], semw
        )
    for wdesc in writes:
        if wdesc is not None:
            wdesc.wait()


_gather_call = functools.partial(
    pl.kernel,
    mesh=plsc.VectorSubcoreMesh(
        core_axis_name="c", subcore_axis_name="s", num_cores=_NC, num_subcores=_NS
    ),
    out_type=jax.ShapeDtypeStruct((_B, D), jnp.float32),
    scratch_types=[
        pltpu.VMEM((_BPW,), jnp.int32),
        pltpu.VMEM((2, _GCH, D), jnp.float32),
        pltpu.VMEM((16,), jnp.float32),
        pltpu.VMEM((16,), jnp.float32),
        pltpu.SemaphoreType.DMA,
        pltpu.SemaphoreType.DMA,
    ],
    compiler_params=pltpu.CompilerParams(use_tc_tiling_on_sc=False),
)(_gather_body)


def kernel(weight, x):
    wt = weight.T
    slabs, scale, svec, ivec = _slab_call(wt)
    tail = _tail_call(wt)
    table = _transpose_call(slabs, tail)
    tview = table.reshape(_GRID_S * _OR * 128 // D, D)
    xf = x.reshape(-1)
    out = _gather_call(xf, tview, svec, ivec)
    return out.reshape(x.shape + (D,)), scale


# speedup vs baseline: 3.9871x; 2.0723x over previous
import functools

import jax
import jax.numpy as jnp
from jax import lax
from jax.experimental import pallas as pl
from jax.experimental.pallas import tpu as pltpu
from jax.experimental.pallas import tpu_sc as plsc

V = 1_000_000
D = 32
N_LEVELS = 127.0
_MAGIC = 1.5 * 2.0 ** 23

_CUT = 999_936
_TAIL = V - _CUT
_TROWS = V * D // 128
_TAILR = _TAIL * D // 128

_SC_ = 512
_NSLAB = _CUT // _SC_
_SPB = 8
_GRID_B = 245
_NSLABT = _GRID_B * _SPB
_GRID_S = _NSLAB + 1


def _slab_body(wt_ref, slab_ref, scale_ref, svec_ref, ivec_ref, acc_ref):
    i = pl.program_id(0)
    w = wt_ref[...]
    for k in range(_SPB):
        slab_ref[k] = w[:, k * _SC_:(k + 1) * _SC_]

    @pl.when(i == 0)
    def _():
        acc_ref[0] = jnp.max(jnp.abs(w))

    @pl.when(jnp.logical_and(i > 0, i < _GRID_B - 1))
    def _():
        acc_ref[0] = jnp.maximum(acc_ref[0], jnp.max(jnp.abs(w)))

    @pl.when(i == _GRID_B - 1)
    def _():
        col = i * (_SPB * _SC_) + lax.broadcasted_iota(
            jnp.int32, (D, _SPB * _SC_), 1)
        m = jnp.max(jnp.where(col < V, jnp.abs(w), 0.0))
        s = jnp.maximum(jnp.maximum(acc_ref[0], m), 1e-8) / N_LEVELS
        scale_ref[0] = s
        for k in range(16):
            svec_ref[k] = s
            ivec_ref[k] = 1.0 / s


_slab_call = pl.pallas_call(
    _slab_body,
    grid=(_GRID_B,),
    in_specs=[pl.BlockSpec((D, _SPB * _SC_), lambda i: (0, i))],
    out_specs=[
        pl.BlockSpec((_SPB, D, _SC_), lambda i: (i, 0, 0)),
        pl.BlockSpec(memory_space=pltpu.SMEM),
        pl.BlockSpec(memory_space=pltpu.SMEM),
        pl.BlockSpec(memory_space=pltpu.SMEM),
    ],
    out_shape=[
        jax.ShapeDtypeStruct((_NSLABT, D, _SC_), jnp.float32),
        jax.ShapeDtypeStruct((1,), jnp.float32),
        jax.ShapeDtypeStruct((16,), jnp.float32),
        jax.ShapeDtypeStruct((16,), jnp.float32),
    ],
    scratch_shapes=[pltpu.SMEM((1,), jnp.float32)],
)

_C61 = 16_384
_TOFF = _CUT - 61 * _C61


def _tail_body(wt_ref, out_ref):
    t = wt_ref[:, _TOFF:_TOFF + _TAIL]
    r = lax.broadcasted_iota(jnp.int32, (_TAILR, _TAIL), 0)
    c = lax.broadcasted_iota(jnp.int32, (_TAILR, _TAIL), 1)
    outs = []
    for k in range(4):
        g = (c == 4 * r + k).astype(jnp.float32)
        outs.append(
            lax.dot_general(g, t, (((1,), (1,)), ((), ())),
                            preferred_element_type=jnp.float32))
    out_ref[...] = jnp.concatenate(outs, axis=1)


_tail_call = pl.pallas_call(
    _tail_body,
    grid=(1,),
    in_specs=[pl.BlockSpec((D, _C61), lambda i: (0, 61))],
    out_specs=pl.BlockSpec((_TAILR, 128), lambda i: (0, 0)),
    out_shape=jax.ShapeDtypeStruct((_TAILR, 128), jnp.float32),
)

_NC, _NS = 2, 16
_NW = _NC * _NS
_OR = _SC_ * D // 128
_CPW = -(-_NSLAB // _NW)


def _transpose_body(slab_ref, tail_ref, out_ref, in_v, out_v, tail_v,
                    semi, semo):
    cc = lax.axis_index("c")
    ss = lax.axis_index("s")
    wid = ss * _NC + cc
    iota = lax.iota(jnp.int32, 16)

    def slab_of(slot):
        return slot * _NW + wid

    def in_copy(slot, p):
        return pltpu.make_async_copy(
            slab_ref.at[slab_of(slot)], in_v.at[p], semi)

    def out_copy(slot, p):
        return pltpu.make_async_copy(
            out_v.at[p], out_ref.at[slab_of(slot)], semo)

    in_copy(0, 0).start()

    @pl.when(wid == 0)
    def _():
        pltpu.sync_copy(tail_ref, tail_v)
        pltpu.sync_copy(tail_v, out_ref.at[_NSLAB, pl.ds(0, _TAILR), :])

    def slot_body(slot, carry):
        p = lax.rem(slot, 2)

        @pl.when(slab_of(slot + 1) < _NSLAB)
        def _():
            in_copy(slot + 1, 1 - p).start()

        @pl.when(jnp.logical_and(slot >= 2, slab_of(slot - 2) < _NSLAB))
        def _():
            out_copy(slot - 2, p).wait()

        @pl.when(slab_of(slot) < _NSLAB)
        def _():
            in_copy(slot, p).wait()
            for d in range(D):
                dvec = lax.bitwise_and(d + iota, 31)
                fbase = 32 * iota + dvec

                @plsc.parallel_loop(0, _SC_ // 16, unroll=8)
                def _(g, dvec=dvec, fbase=fbase):
                    cvec = 16 * g + iota
                    v = plsc.load_gather(in_v.at[p], [dvec, cvec])
                    flat = 512 * g + fbase
                    plsc.store_scatter(
                        out_v.at[p],
                        [lax.shift_right_logical(flat, 7),
                         lax.bitwise_and(flat, 127)],
                        v)
            out_copy(slot, p).start()

        return carry

    lax.fori_loop(0, _CPW, slot_body, 0)

    for s in (_CPW - 2, _CPW - 1):

        @pl.when(slab_of(s) < _NSLAB)
        def _(s=s):
            out_copy(s, s % 2).wait()


_transpose_call = functools.partial(
    pl.kernel,
    mesh=plsc.VectorSubcoreMesh(
        core_axis_name="c", subcore_axis_name="s", num_cores=_NC, num_subcores=_NS
    ),
    out_type=jax.ShapeDtypeStruct((_GRID_S, _OR, 128), jnp.float32),
    scratch_types=[
        pltpu.VMEM((2, D, _SC_), jnp.float32),
        pltpu.VMEM((2, _OR, 128), jnp.float32),
        pltpu.VMEM((_TAILR, 128), jnp.float32),
        pltpu.SemaphoreType.DMA,
        pltpu.SemaphoreType.DMA,
    ],
    compiler_params=pltpu.CompilerParams(
        use_tc_tiling_on_sc=True, needs_layout_passes=False
    ),
)(_transpose_body)

_B = 16384 * 20
_BPW = _B // _NW
_CH = 128
_K = 8
_GCH = _CH * _K
_NG = _BPW // _GCH


def _gather_body(x_ref, w_ref, svec_ref, ivec_ref, out_ref,
                 idx_v, rows_v, sv_v, iv_v, semg, semw):
    cc = lax.axis_index("c")
    ss = lax.axis_index("s")
    wid = ss * _NC + cc
    base = wid * _BPW
    pltpu.sync_copy(svec_ref, sv_v)
    pltpu.sync_copy(ivec_ref, iv_v)
    svec = sv_v[...]
    ivec = iv_v[...]
    pltpu.sync_copy(x_ref.at[pl.ds(base, _BPW)], idx_v)

    def fire(g, p):
        return [
            pltpu.async_copy(
                w_ref.at[idx_v.at[pl.ds(g * _GCH + j * _CH, _CH)]],
                rows_v.at[p, pl.ds(j * _CH, _CH)],
                semg,
            )
            for j in range(_K)
        ]

    descs = fire(0, 0)
    writes = [None, None]
    for g in range(_NG):
        p = g % 2
        for d_ in descs:
            d_.wait()
        if g + 1 < _NG:
            if writes[1 - p] is not None:
                writes[1 - p].wait()
            descs = fire(g + 1, 1 - p)

        def qbody(r, c2):
            for h in (0, 1):
                v = rows_v[p, r, pl.ds(16 * h, 16)]
                q = (v * ivec + _MAGIC) - _MAGIC
                rows_v[p, r, pl.ds(16 * h, 16)] = (
                    jnp.minimum(q, N_LEVELS - 1.0) * svec)
            return c2

        lax.fori_loop(0, _GCH, qbody, 0, unroll=4)
        writes[p] = pltpu.async_copy(
            rows_v.at[p], out_ref.at[pl.ds(base + g * _GCH, _GCH)], semw
        )
    for wdesc in writes:
        if wdesc is not None:
            wdesc.wait()


_gather_call = functools.partial(
    pl.kernel,
    mesh=plsc.VectorSubcoreMesh(
        core_axis_name="c", subcore_axis_name="s", num_cores=_NC, num_subcores=_NS
    ),
    out_type=jax.ShapeDtypeStruct((_B, D), jnp.float32),
    scratch_types=[
        pltpu.VMEM((_BPW,), jnp.int32),
        pltpu.VMEM((2, _GCH, D), jnp.float32),
        pltpu.VMEM((16,), jnp.float32),
        pltpu.VMEM((16,), jnp.float32),
        pltpu.SemaphoreType.DMA,
        pltpu.SemaphoreType.DMA,
    ],
    compiler_params=pltpu.CompilerParams(use_tc_tiling_on_sc=False),
)(_gather_body)


def kernel(weight, x):
    wt = weight.T
    slabs, scale, svec, ivec = _slab_call(wt)
    tail = _tail_call(wt)
    table = _transpose_call(slabs, tail)
    tview = table.reshape(_GRID_S * _OR * 128 // D, D)
    xf = x.reshape(-1)
    out = _gather_call(xf, tview, svec, ivec)
    return out.reshape(x.shape + (D,)), scale


# speedup vs baseline: 4.4674x; 1.1204x over previous
import functools

import jax
import jax.numpy as jnp
from jax import lax
from jax.experimental import pallas as pl
from jax.experimental.pallas import tpu as pltpu
from jax.experimental.pallas import tpu_sc as plsc

V = 1_000_000
D = 32
N_LEVELS = 127.0
_MAGIC = 1.5 * 2.0 ** 23

_CUT = 999_936
_TAIL = V - _CUT
_TROWS = V * D // 128
_TAILR = _TAIL * D // 128

_SC_ = 512
_NSLAB = _CUT // _SC_
_SPB = 16
_GRID_B = 123
_NSLABT = _GRID_B * _SPB
_GRID_S = _NSLAB + 1


def _slab_body(wt_ref, slab_ref, scale_ref, svec_ref, ivec_ref, acc_ref):
    i = pl.program_id(0)
    w = wt_ref[...]
    for k in range(_SPB):
        slab_ref[k] = w[:, k * _SC_:(k + 1) * _SC_]

    @pl.when(i == 0)
    def _():
        acc_ref[0] = jnp.max(jnp.abs(w))

    @pl.when(jnp.logical_and(i > 0, i < _GRID_B - 1))
    def _():
        acc_ref[0] = jnp.maximum(acc_ref[0], jnp.max(jnp.abs(w)))

    @pl.when(i == _GRID_B - 1)
    def _():
        col = i * (_SPB * _SC_) + lax.broadcasted_iota(
            jnp.int32, (D, _SPB * _SC_), 1)
        m = jnp.max(jnp.where(col < V, jnp.abs(w), 0.0))
        s = jnp.maximum(jnp.maximum(acc_ref[0], m), 1e-8) / N_LEVELS
        scale_ref[0] = s
        for k in range(16):
            svec_ref[k] = s
            ivec_ref[k] = 1.0 / s


_slab_call = pl.pallas_call(
    _slab_body,
    grid=(_GRID_B,),
    in_specs=[pl.BlockSpec((D, _SPB * _SC_), lambda i: (0, i))],
    out_specs=[
        pl.BlockSpec((_SPB, D, _SC_), lambda i: (i, 0, 0)),
        pl.BlockSpec(memory_space=pltpu.SMEM),
        pl.BlockSpec(memory_space=pltpu.SMEM),
        pl.BlockSpec(memory_space=pltpu.SMEM),
    ],
    out_shape=[
        jax.ShapeDtypeStruct((_NSLABT, D, _SC_), jnp.float32),
        jax.ShapeDtypeStruct((1,), jnp.float32),
        jax.ShapeDtypeStruct((16,), jnp.float32),
        jax.ShapeDtypeStruct((16,), jnp.float32),
    ],
    scratch_shapes=[pltpu.SMEM((1,), jnp.float32)],
)

_C61 = 16_384
_TOFF = _CUT - 61 * _C61


def _tail_body(wt_ref, out_ref):
    t = wt_ref[:, _TOFF:_TOFF + _TAIL]
    r = lax.broadcasted_iota(jnp.int32, (_TAILR, _TAIL), 0)
    c = lax.broadcasted_iota(jnp.int32, (_TAILR, _TAIL), 1)
    outs = []
    for k in range(4):
        g = (c == 4 * r + k).astype(jnp.float32)
        outs.append(
            lax.dot_general(g, t, (((1,), (1,)), ((), ())),
                            preferred_element_type=jnp.float32))
    out_ref[...] = jnp.concatenate(outs, axis=1)


_tail_call = pl.pallas_call(
    _tail_body,
    grid=(1,),
    in_specs=[pl.BlockSpec((D, _C61), lambda i: (0, 61))],
    out_specs=pl.BlockSpec((_TAILR, 128), lambda i: (0, 0)),
    out_shape=jax.ShapeDtypeStruct((_TAILR, 128), jnp.float32),
)

_NC, _NS = 2, 16
_NW = _NC * _NS
_OR = _SC_ * D // 128
_CPW = -(-_NSLAB // _NW)


def _transpose_body(slab_ref, tail_ref, out_ref, in_v, out_v, tail_v,
                    semi, semo):
    cc = lax.axis_index("c")
    ss = lax.axis_index("s")
    wid = ss * _NC + cc
    iota = lax.iota(jnp.int32, 16)

    def slab_of(slot):
        return slot * _NW + wid

    def in_copy(slot, p):
        return pltpu.make_async_copy(
            slab_ref.at[slab_of(slot)], in_v.at[p], semi)

    def out_copy(slot, p):
        return pltpu.make_async_copy(
            out_v.at[p], out_ref.at[slab_of(slot)], semo)

    in_copy(0, 0).start()

    @pl.when(wid == 0)
    def _():
        pltpu.sync_copy(tail_ref, tail_v)
        pltpu.sync_copy(tail_v, out_ref.at[_NSLAB, pl.ds(0, _TAILR), :])

    def slot_body(slot, carry):
        p = lax.rem(slot, 2)

        @pl.when(slab_of(slot + 1) < _NSLAB)
        def _():
            in_copy(slot + 1, 1 - p).start()

        @pl.when(jnp.logical_and(slot >= 2, slab_of(slot - 2) < _NSLAB))
        def _():
            out_copy(slot - 2, p).wait()

        @pl.when(slab_of(slot) < _NSLAB)
        def _():
            in_copy(slot, p).wait()
            for d in range(D):
                dvec = lax.bitwise_and(d + iota, 31)
                fbase = 32 * iota + dvec

                @plsc.parallel_loop(0, _SC_ // 16, unroll=8)
                def _(g, dvec=dvec, fbase=fbase):
                    cvec = 16 * g + iota
                    v = plsc.load_gather(in_v.at[p], [dvec, cvec])
                    flat = 512 * g + fbase
                    plsc.store_scatter(
                        out_v.at[p],
                        [lax.shift_right_logical(flat, 7),
                         lax.bitwise_and(flat, 127)],
                        v)
            out_copy(slot, p).start()

        return carry

    lax.fori_loop(0, _CPW, slot_body, 0)

    for s in (_CPW - 2, _CPW - 1):

        @pl.when(slab_of(s) < _NSLAB)
        def _(s=s):
            out_copy(s, s % 2).wait()


_transpose_call = functools.partial(
    pl.kernel,
    mesh=plsc.VectorSubcoreMesh(
        core_axis_name="c", subcore_axis_name="s", num_cores=_NC, num_subcores=_NS
    ),
    out_type=jax.ShapeDtypeStruct((_GRID_S, _OR, 128), jnp.float32),
    scratch_types=[
        pltpu.VMEM((2, D, _SC_), jnp.float32),
        pltpu.VMEM((2, _OR, 128), jnp.float32),
        pltpu.VMEM((_TAILR, 128), jnp.float32),
        pltpu.SemaphoreType.DMA,
        pltpu.SemaphoreType.DMA,
    ],
    compiler_params=pltpu.CompilerParams(
        use_tc_tiling_on_sc=True, needs_layout_passes=False
    ),
)(_transpose_body)

_B = 16384 * 20
_BPW = _B // _NW
_CH = 128
_K = 8
_GCH = _CH * _K
_NG = _BPW // _GCH


def _gather_body(x_ref, w_ref, svec_ref, ivec_ref, out_ref,
                 idx_v, rows_v, sv_v, iv_v, semg, semw):
    cc = lax.axis_index("c")
    ss = lax.axis_index("s")
    wid = ss * _NC + cc
    base = wid * _BPW
    pltpu.sync_copy(svec_ref, sv_v)
    pltpu.sync_copy(ivec_ref, iv_v)
    svec = sv_v[...]
    ivec = iv_v[...]
    pltpu.sync_copy(x_ref.at[pl.ds(base, _BPW)], idx_v)

    def fire(g, p):
        return [
            pltpu.async_copy(
                w_ref.at[idx_v.at[pl.ds(g * _GCH + j * _CH, _CH)]],
                rows_v.at[p, pl.ds(j * _CH, _CH)],
                semg,
            )
            for j in range(_K)
        ]

    descs = fire(0, 0)
    writes = [None, None]
    for g in range(_NG):
        p = g % 2
        for d_ in descs:
            d_.wait()
        if g + 1 < _NG:
            if writes[1 - p] is not None:
                writes[1 - p].wait()
            descs = fire(g + 1, 1 - p)

        def qbody(r, c2):
            for h in (0, 1):
                v = rows_v[p, r, pl.ds(16 * h, 16)]
                q = (v * ivec + _MAGIC) - _MAGIC
                rows_v[p, r, pl.ds(16 * h, 16)] = (
                    jnp.minimum(q, N_LEVELS - 1.0) * svec)
            return c2

        lax.fori_loop(0, _GCH, qbody, 0, unroll=4)
        writes[p] = pltpu.async_copy(
            rows_v.at[p], out_ref.at[pl.ds(base + g * _GCH, _GCH)], semw
        )
    for wdesc in writes:
        if wdesc is not None:
            wdesc.wait()


_gather_call = functools.partial(
    pl.kernel,
    mesh=plsc.VectorSubcoreMesh(
        core_axis_name="c", subcore_axis_name="s", num_cores=_NC, num_subcores=_NS
    ),
    out_type=jax.ShapeDtypeStruct((_B, D), jnp.float32),
    scratch_types=[
        pltpu.VMEM((_BPW,), jnp.int32),
        pltpu.VMEM((2, _GCH, D), jnp.float32),
        pltpu.VMEM((16,), jnp.float32),
        pltpu.VMEM((16,), jnp.float32),
        pltpu.SemaphoreType.DMA,
        pltpu.SemaphoreType.DMA,
    ],
    compiler_params=pltpu.CompilerParams(use_tc_tiling_on_sc=False),
)(_gather_body)


def kernel(weight, x):
    wt = weight.T
    slabs, scale, svec, ivec = _slab_call(wt)
    tail = _tail_call(wt)
    table = _transpose_call(slabs, tail)
    tview = table.reshape(_GRID_S * _OR * 128 // D, D)
    xf = x.reshape(-1)
    out = _gather_call(xf, tview, svec, ivec)
    return out.reshape(x.shape + (D,)), scale


# speedup vs baseline: 4.8060x; 1.0758x over previous
import functools

import jax
import jax.numpy as jnp
from jax import lax
from jax.experimental import pallas as pl
from jax.experimental.pallas import tpu as pltpu
from jax.experimental.pallas import tpu_sc as plsc

V = 1_000_000
D = 32
N_LEVELS = 127.0
_MAGIC = 1.5 * 2.0 ** 23

_CUT = 999_936
_TAIL = V - _CUT
_TROWS = V * D // 128
_TAILR = _TAIL * D // 128

_SC_ = 512
_NSLAB = _CUT // _SC_
_SPB = 32
_GRID_B = 62
_NSLABT = _GRID_B * _SPB
_GRID_S = _NSLAB + 1


def _slab_body(wt_ref, slab_ref, scale_ref, svec_ref, ivec_ref, acc_ref):
    i = pl.program_id(0)
    w = wt_ref[...]
    for k in range(_SPB):
        slab_ref[k] = w[:, k * _SC_:(k + 1) * _SC_]

    @pl.when(i == 0)
    def _():
        acc_ref[0] = jnp.max(jnp.abs(w))

    @pl.when(jnp.logical_and(i > 0, i < _GRID_B - 1))
    def _():
        acc_ref[0] = jnp.maximum(acc_ref[0], jnp.max(jnp.abs(w)))

    @pl.when(i == _GRID_B - 1)
    def _():
        col = i * (_SPB * _SC_) + lax.broadcasted_iota(
            jnp.int32, (D, _SPB * _SC_), 1)
        m = jnp.max(jnp.where(col < V, jnp.abs(w), 0.0))
        s = jnp.maximum(jnp.maximum(acc_ref[0], m), 1e-8) / N_LEVELS
        scale_ref[0] = s
        for k in range(16):
            svec_ref[k] = s
            ivec_ref[k] = 1.0 / s


_slab_call = pl.pallas_call(
    _slab_body,
    grid=(_GRID_B,),
    in_specs=[pl.BlockSpec((D, _SPB * _SC_), lambda i: (0, i))],
    out_specs=[
        pl.BlockSpec((_SPB, D, _SC_), lambda i: (i, 0, 0)),
        pl.BlockSpec(memory_space=pltpu.SMEM),
        pl.BlockSpec(memory_space=pltpu.SMEM),
        pl.BlockSpec(memory_space=pltpu.SMEM),
    ],
    out_shape=[
        jax.ShapeDtypeStruct((_NSLABT, D, _SC_), jnp.float32),
        jax.ShapeDtypeStruct((1,), jnp.float32),
        jax.ShapeDtypeStruct((16,), jnp.float32),
        jax.ShapeDtypeStruct((16,), jnp.float32),
    ],
    scratch_shapes=[pltpu.SMEM((1,), jnp.float32)],
)

_C61 = 16_384
_TOFF = _CUT - 61 * _C61


def _tail_body(wt_ref, out_ref):
    t = wt_ref[:, _TOFF:_TOFF + _TAIL]
    r = lax.broadcasted_iota(jnp.int32, (_TAILR, _TAIL), 0)
    c = lax.broadcasted_iota(jnp.int32, (_TAILR, _TAIL), 1)
    outs = []
    for k in range(4):
        g = (c == 4 * r + k).astype(jnp.float32)
        outs.append(
            lax.dot_general(g, t, (((1,), (1,)), ((), ())),
                            preferred_element_type=jnp.float32))
    out_ref[...] = jnp.concatenate(outs, axis=1)


_tail_call = pl.pallas_call(
    _tail_body,
    grid=(1,),
    in_specs=[pl.BlockSpec((D, _C61), lambda i: (0, 61))],
    out_specs=pl.BlockSpec((_TAILR, 128), lambda i: (0, 0)),
    out_shape=jax.ShapeDtypeStruct((_TAILR, 128), jnp.float32),
)

_NC, _NS = 2, 16
_NW = _NC * _NS
_OR = _SC_ * D // 128
_CPW = -(-_NSLAB // _NW)


def _transpose_body(slab_ref, tail_ref, out_ref, in_v, out_v, tail_v,
                    semi, semo):
    cc = lax.axis_index("c")
    ss = lax.axis_index("s")
    wid = ss * _NC + cc
    iota = lax.iota(jnp.int32, 16)

    def slab_of(slot):
        return slot * _NW + wid

    def in_copy(slot, p):
        return pltpu.make_async_copy(
            slab_ref.at[slab_of(slot)], in_v.at[p], semi)

    def out_copy(slot, p):
        return pltpu.make_async_copy(
            out_v.at[p], out_ref.at[slab_of(slot)], semo)

    in_copy(0, 0).start()

    @pl.when(wid == 0)
    def _():
        pltpu.sync_copy(tail_ref, tail_v)
        pltpu.sync_copy(tail_v, out_ref.at[_NSLAB, pl.ds(0, _TAILR), :])

    def slot_body(slot, carry):
        p = lax.rem(slot, 2)

        @pl.when(slab_of(slot + 1) < _NSLAB)
        def _():
            in_copy(slot + 1, 1 - p).start()

        @pl.when(jnp.logical_and(slot >= 2, slab_of(slot - 2) < _NSLAB))
        def _():
            out_copy(slot - 2, p).wait()

        @pl.when(slab_of(slot) < _NSLAB)
        def _():
            in_copy(slot, p).wait()
            for d in range(D):
                dvec = lax.bitwise_and(d + iota, 31)
                fbase = 32 * iota + dvec

                @plsc.parallel_loop(0, _SC_ // 16, unroll=8)
                def _(g, dvec=dvec, fbase=fbase):
                    cvec = 16 * g + iota
                    v = plsc.load_gather(in_v.at[p], [dvec, cvec])
                    flat = 512 * g + fbase
                    plsc.store_scatter(
                        out_v.at[p],
                        [lax.shift_right_logical(flat, 7),
                         lax.bitwise_and(flat, 127)],
                        v)
            out_copy(slot, p).start()

        return carry

    lax.fori_loop(0, _CPW, slot_body, 0)

    for s in (_CPW - 2, _CPW - 1):

        @pl.when(slab_of(s) < _NSLAB)
        def _(s=s):
            out_copy(s, s % 2).wait()


_transpose_call = functools.partial(
    pl.kernel,
    mesh=plsc.VectorSubcoreMesh(
        core_axis_name="c", subcore_axis_name="s", num_cores=_NC, num_subcores=_NS
    ),
    out_type=jax.ShapeDtypeStruct((_GRID_S, _OR, 128), jnp.float32),
    scratch_types=[
        pltpu.VMEM((2, D, _SC_), jnp.float32),
        pltpu.VMEM((2, _OR, 128), jnp.float32),
        pltpu.VMEM((_TAILR, 128), jnp.float32),
        pltpu.SemaphoreType.DMA,
        pltpu.SemaphoreType.DMA,
    ],
    compiler_params=pltpu.CompilerParams(
        use_tc_tiling_on_sc=True, needs_layout_passes=False
    ),
)(_transpose_body)

_B = 16384 * 20
_BPW = _B // _NW
_CH = 128
_K = 8
_GCH = _CH * _K
_NG = _BPW // _GCH


def _gather_body(x_ref, w_ref, svec_ref, ivec_ref, out_ref,
                 idx_v, rows_v, sv_v, iv_v, semg, semw):
    cc = lax.axis_index("c")
    ss = lax.axis_index("s")
    wid = ss * _NC + cc
    base = wid * _BPW
    pltpu.sync_copy(svec_ref, sv_v)
    pltpu.sync_copy(ivec_ref, iv_v)
    svec = sv_v[...]
    ivec = iv_v[...]
    pltpu.sync_copy(x_ref.at[pl.ds(base, _BPW)], idx_v)

    def fire(g, p):
        return [
            pltpu.async_copy(
                w_ref.at[idx_v.at[pl.ds(g * _GCH + j * _CH, _CH)]],
                rows_v.at[p, pl.ds(j * _CH, _CH)],
                semg,
            )
            for j in range(_K)
        ]

    descs = fire(0, 0)
    writes = [None, None]
    for g in range(_NG):
        p = g % 2
        for d_ in descs:
            d_.wait()
        if g + 1 < _NG:
            if writes[1 - p] is not None:
                writes[1 - p].wait()
            descs = fire(g + 1, 1 - p)

        def qbody(r, c2):
            for h in (0, 1):
                v = rows_v[p, r, pl.ds(16 * h, 16)]
                q = (v * ivec + _MAGIC) - _MAGIC
                rows_v[p, r, pl.ds(16 * h, 16)] = (
                    jnp.minimum(q, N_LEVELS - 1.0) * svec)
            return c2

        lax.fori_loop(0, _GCH, qbody, 0, unroll=4)
        writes[p] = pltpu.async_copy(
            rows_v.at[p], out_ref.at[pl.ds(base + g * _GCH, _GCH)], semw
        )
    for wdesc in writes:
        if wdesc is not None:
            wdesc.wait()


_gather_call = functools.partial(
    pl.kernel,
    mesh=plsc.VectorSubcoreMesh(
        core_axis_name="c", subcore_axis_name="s", num_cores=_NC, num_subcores=_NS
    ),
    out_type=jax.ShapeDtypeStruct((_B, D), jnp.float32),
    scratch_types=[
        pltpu.VMEM((_BPW,), jnp.int32),
        pltpu.VMEM((2, _GCH, D), jnp.float32),
        pltpu.VMEM((16,), jnp.float32),
        pltpu.VMEM((16,), jnp.float32),
        pltpu.SemaphoreType.DMA,
        pltpu.SemaphoreType.DMA,
    ],
    compiler_params=pltpu.CompilerParams(use_tc_tiling_on_sc=False),
)(_gather_body)


def kernel(weight, x):
    wt = weight.T
    slabs, scale, svec, ivec = _slab_call(wt)
    tail = _tail_call(wt)
    table = _transpose_call(slabs, tail)
    tview = table.reshape(_GRID_S * _OR * 128 // D, D)
    xf = x.reshape(-1)
    out = _gather_call(xf, tview, svec, ivec)
    return out.reshape(x.shape + (D,)), scale


# speedup vs baseline: 4.9478x; 1.0295x over previous
import functools

import jax
import jax.numpy as jnp
from jax import lax
from jax.experimental import pallas as pl
from jax.experimental.pallas import tpu as pltpu
from jax.experimental.pallas import tpu_sc as plsc

V = 1_000_000
D = 32
N_LEVELS = 127.0
_MAGIC = 1.5 * 2.0 ** 23

_CUT = 999_936
_TAIL = V - _CUT
_TROWS = V * D // 128
_TAILR = _TAIL * D // 128

_SC_ = 512
_NSLAB = _CUT // _SC_
_SPB = 64
_GRID_B = 31
_NSLABT = _GRID_B * _SPB
_GRID_S = _NSLAB + 1


def _slab_body(wt_ref, slab_ref, scale_ref, svec_ref, ivec_ref, acc_ref):
    i = pl.program_id(0)
    w = wt_ref[...]
    for k in range(_SPB):
        slab_ref[k] = w[:, k * _SC_:(k + 1) * _SC_]

    @pl.when(i == 0)
    def _():
        acc_ref[0] = jnp.max(jnp.abs(w))

    @pl.when(jnp.logical_and(i > 0, i < _GRID_B - 1))
    def _():
        acc_ref[0] = jnp.maximum(acc_ref[0], jnp.max(jnp.abs(w)))

    @pl.when(i == _GRID_B - 1)
    def _():
        col = i * (_SPB * _SC_) + lax.broadcasted_iota(
            jnp.int32, (D, _SPB * _SC_), 1)
        m = jnp.max(jnp.where(col < V, jnp.abs(w), 0.0))
        s = jnp.maximum(jnp.maximum(acc_ref[0], m), 1e-8) / N_LEVELS
        scale_ref[0] = s
        for k in range(16):
            svec_ref[k] = s
            ivec_ref[k] = 1.0 / s


_slab_call = pl.pallas_call(
    _slab_body,
    grid=(_GRID_B,),
    in_specs=[pl.BlockSpec((D, _SPB * _SC_), lambda i: (0, i))],
    out_specs=[
        pl.BlockSpec((_SPB, D, _SC_), lambda i: (i, 0, 0)),
        pl.BlockSpec(memory_space=pltpu.SMEM),
        pl.BlockSpec(memory_space=pltpu.SMEM),
        pl.BlockSpec(memory_space=pltpu.SMEM),
    ],
    out_shape=[
        jax.ShapeDtypeStruct((_NSLABT, D, _SC_), jnp.float32),
        jax.ShapeDtypeStruct((1,), jnp.float32),
        jax.ShapeDtypeStruct((16,), jnp.float32),
        jax.ShapeDtypeStruct((16,), jnp.float32),
    ],
    scratch_shapes=[pltpu.SMEM((1,), jnp.float32)],
)

_C61 = 16_384
_TOFF = _CUT - 61 * _C61


def _tail_body(wt_ref, out_ref):
    t = wt_ref[:, _TOFF:_TOFF + _TAIL]
    r = lax.broadcasted_iota(jnp.int32, (_TAILR, _TAIL), 0)
    c = lax.broadcasted_iota(jnp.int32, (_TAILR, _TAIL), 1)
    outs = []
    for k in range(4):
        g = (c == 4 * r + k).astype(jnp.float32)
        outs.append(
            lax.dot_general(g, t, (((1,), (1,)), ((), ())),
                            preferred_element_type=jnp.float32))
    out_ref[...] = jnp.concatenate(outs, axis=1)


_tail_call = pl.pallas_call(
    _tail_body,
    grid=(1,),
    in_specs=[pl.BlockSpec((D, _C61), lambda i: (0, 61))],
    out_specs=pl.BlockSpec((_TAILR, 128), lambda i: (0, 0)),
    out_shape=jax.ShapeDtypeStruct((_TAILR, 128), jnp.float32),
)

_NC, _NS = 2, 16
_NW = _NC * _NS
_OR = _SC_ * D // 128
_CPW = -(-_NSLAB // _NW)


def _transpose_body(slab_ref, tail_ref, out_ref, in_v, out_v, tail_v,
                    semi, semo):
    cc = lax.axis_index("c")
    ss = lax.axis_index("s")
    wid = ss * _NC + cc
    iota = lax.iota(jnp.int32, 16)

    def slab_of(slot):
        return slot * _NW + wid

    def in_copy(slot, p):
        return pltpu.make_async_copy(
            slab_ref.at[slab_of(slot)], in_v.at[p], semi)

    def out_copy(slot, p):
        return pltpu.make_async_copy(
            out_v.at[p], out_ref.at[slab_of(slot)], semo)

    in_copy(0, 0).start()

    @pl.when(wid == 0)
    def _():
        pltpu.sync_copy(tail_ref, tail_v)
        pltpu.sync_copy(tail_v, out_ref.at[_NSLAB, pl.ds(0, _TAILR), :])

    def slot_body(slot, carry):
        p = lax.rem(slot, 2)

        @pl.when(slab_of(slot + 1) < _NSLAB)
        def _():
            in_copy(slot + 1, 1 - p).start()

        @pl.when(jnp.logical_and(slot >= 2, slab_of(slot - 2) < _NSLAB))
        def _():
            out_copy(slot - 2, p).wait()

        @pl.when(slab_of(slot) < _NSLAB)
        def _():
            in_copy(slot, p).wait()
            for d in range(D):
                dvec = lax.bitwise_and(d + iota, 31)
                fbase = 32 * iota + dvec

                @plsc.parallel_loop(0, _SC_ // 16, unroll=8)
                def _(g, dvec=dvec, fbase=fbase):
                    cvec = 16 * g + iota
                    v = plsc.load_gather(in_v.at[p], [dvec, cvec])
                    flat = 512 * g + fbase
                    plsc.store_scatter(
                        out_v.at[p],
                        [lax.shift_right_logical(flat, 7),
                         lax.bitwise_and(flat, 127)],
                        v)
            out_copy(slot, p).start()

        return carry

    lax.fori_loop(0, _CPW, slot_body, 0)

    for s in (_CPW - 2, _CPW - 1):

        @pl.when(slab_of(s) < _NSLAB)
        def _(s=s):
            out_copy(s, s % 2).wait()


_transpose_call = functools.partial(
    pl.kernel,
    mesh=plsc.VectorSubcoreMesh(
        core_axis_name="c", subcore_axis_name="s", num_cores=_NC, num_subcores=_NS
    ),
    out_type=jax.ShapeDtypeStruct((_GRID_S, _OR, 128), jnp.float32),
    scratch_types=[
        pltpu.VMEM((2, D, _SC_), jnp.float32),
        pltpu.VMEM((2, _OR, 128), jnp.float32),
        pltpu.VMEM((_TAILR, 128), jnp.float32),
        pltpu.SemaphoreType.DMA,
        pltpu.SemaphoreType.DMA,
    ],
    compiler_params=pltpu.CompilerParams(
        use_tc_tiling_on_sc=True, needs_layout_passes=False
    ),
)(_transpose_body)

_B = 16384 * 20
_BPW = _B // _NW
_CH = 128
_K = 8
_GCH = _CH * _K
_NG = _BPW // _GCH


def _gather_body(x_ref, w_ref, svec_ref, ivec_ref, out_ref,
                 idx_v, rows_v, sv_v, iv_v, semg, semw):
    cc = lax.axis_index("c")
    ss = lax.axis_index("s")
    wid = ss * _NC + cc
    base = wid * _BPW
    pltpu.sync_copy(svec_ref, sv_v)
    pltpu.sync_copy(ivec_ref, iv_v)
    svec = sv_v[...]
    ivec = iv_v[...]
    pltpu.sync_copy(x_ref.at[pl.ds(base, _BPW)], idx_v)

    def fire(g, p):
        return [
            pltpu.async_copy(
                w_ref.at[idx_v.at[pl.ds(g * _GCH + j * _CH, _CH)]],
                rows_v.at[p, pl.ds(j * _CH, _CH)],
                semg,
            )
            for j in range(_K)
        ]

    descs = fire(0, 0)
    writes = [None, None]
    for g in range(_NG):
        p = g % 2
        for d_ in descs:
            d_.wait()
        if g + 1 < _NG:
            if writes[1 - p] is not None:
                writes[1 - p].wait()
            descs = fire(g + 1, 1 - p)

        def qbody(r, c2):
            for h in (0, 1):
                v = rows_v[p, r, pl.ds(16 * h, 16)]
                q = (v * ivec + _MAGIC) - _MAGIC
                rows_v[p, r, pl.ds(16 * h, 16)] = (
                    jnp.minimum(q, N_LEVELS - 1.0) * svec)
            return c2

        lax.fori_loop(0, _GCH, qbody, 0, unroll=4)
        writes[p] = pltpu.async_copy(
            rows_v.at[p], out_ref.at[pl.ds(base + g * _GCH, _GCH)], semw
        )
    for wdesc in writes:
        if wdesc is not None:
            wdesc.wait()


_gather_call = functools.partial(
    pl.kernel,
    mesh=plsc.VectorSubcoreMesh(
        core_axis_name="c", subcore_axis_name="s", num_cores=_NC, num_subcores=_NS
    ),
    out_type=jax.ShapeDtypeStruct((_B, D), jnp.float32),
    scratch_types=[
        pltpu.VMEM((_BPW,), jnp.int32),
        pltpu.VMEM((2, _GCH, D), jnp.float32),
        pltpu.VMEM((16,), jnp.float32),
        pltpu.VMEM((16,), jnp.float32),
        pltpu.SemaphoreType.DMA,
        pltpu.SemaphoreType.DMA,
    ],
    compiler_params=pltpu.CompilerParams(use_tc_tiling_on_sc=False),
)(_gather_body)


def kernel(weight, x):
    wt = weight.T
    slabs, scale, svec, ivec = _slab_call(wt)
    tail = _tail_call(wt)
    table = _transpose_call(slabs, tail)
    tview = table.reshape(_GRID_S * _OR * 128 // D, D)
    xf = x.reshape(-1)
    out = _gather_call(xf, tview, svec, ivec)
    return out.reshape(x.shape + (D,)), scale
